# Initial kernel scaffold; baseline (speedup 1.0000x reference)
import jax
import jax.numpy as jnp
from jax.experimental import pallas as pl


def kernel(x, edge_index, W_line, b_line, W_g1, a_src1, a_dst1, b_g1, W_l1, b_l1, W_g2, a_src2, a_dst2, b_g2, W_l2, b_l2, W_fc, b_fc, W_out, b_out):
    raise NotImplementedError("write your pallas kernel here")


# speedup vs baseline: 72.1051x; 72.1051x over previous
import functools
import numpy as np
import jax
import jax.numpy as jnp
from jax import lax
from jax.experimental import pallas as pl
from jax.experimental.pallas import tpu as pltpu
from jax.experimental.pallas import tpu_sc as plsc

N = 50000
E = 1600000
D_IN = 1346
H = 4
C = 8
HID = 32

BN = 1000
GRID = N // BN
E_TOT = E + N
NW = 32
CHUNK = 128
EPW = 51584
E_PAD = NW * EPW
NCHUNK = EPW // CHUNK
NP = 50048
ROWS_PT = NP // 16
TW = 48
AW = 32
EW = 16


def _leaky(x):
    return jnp.where(x >= 0, x, 0.2 * x)


def _elu(x):
    return jnp.where(x > 0, x, jnp.exp(jnp.minimum(x, 0.0)) - 1.0)


def _k1_body(x_ref, wl_ref, bl_ref, wg_ref, asf_ref, adf_ref, S_ref,
             h1_ref, ts_ref, td_ref, ta_ref, mx_ref):
    h1 = jnp.maximum(jnp.dot(x_ref[...], wl_ref[...],
                             preferred_element_type=jnp.float32)
                     + bl_ref[...], 0.0)
    h1_ref[...] = h1
    xw = jnp.dot(h1, wg_ref[...], preferred_element_type=jnp.float32)
    a_s = jnp.dot(xw * asf_ref[...], S_ref[...],
                  preferred_element_type=jnp.float32)
    a_d = jnp.dot(xw * adf_ref[...], S_ref[...],
                  preferred_element_type=jnp.float32)
    z = jnp.zeros((BN, 12), jnp.float32)
    ts_ref[...] = jnp.concatenate([a_s, z, xw], axis=1)
    td_ref[...] = jnp.concatenate([a_d, z], axis=1)
    ta_ref[...] = jnp.concatenate([a_s, z], axis=1)
    mx_ref[...] = jnp.concatenate(
        [jnp.max(a_s, axis=0), jnp.max(a_d, axis=0)], axis=0).reshape(1, 1, 8)


def _k1(x, W_line, b_line, W_g1, as_flat, ad_flat, S):
    return pl.pallas_call(
        _k1_body,
        grid=(GRID,),
        in_specs=[
            pl.BlockSpec((BN, D_IN), lambda i: (i, 0)),
            pl.BlockSpec((D_IN, HID), lambda i: (0, 0)),
            pl.BlockSpec((1, HID), lambda i: (0, 0)),
            pl.BlockSpec((HID, HID), lambda i: (0, 0)),
            pl.BlockSpec((1, HID), lambda i: (0, 0)),
            pl.BlockSpec((1, HID), lambda i: (0, 0)),
            pl.BlockSpec((HID, H), lambda i: (0, 0)),
        ],
        out_specs=[
            pl.BlockSpec((BN, HID), lambda i: (i, 0)),
            pl.BlockSpec((BN, TW), lambda i: (i, 0)),
            pl.BlockSpec((BN, 16), lambda i: (i, 0)),
            pl.BlockSpec((BN, 16), lambda i: (i, 0)),
            pl.BlockSpec((1, 1, 8), lambda i: (i, 0, 0)),
        ],
        out_shape=[
            jax.ShapeDtypeStruct((N, HID), jnp.float32),
            jax.ShapeDtypeStruct((N, TW), jnp.float32),
            jax.ShapeDtypeStruct((N, 16), jnp.float32),
            jax.ShapeDtypeStruct((N, 16), jnp.float32),
            jax.ShapeDtypeStruct((GRID, 1, 8), jnp.float32),
        ],
    )(x, W_line, b_line.reshape(1, HID), W_g1,
      as_flat.reshape(1, HID), ad_flat.reshape(1, HID), S)


def _dg(v, idx):
    return lax.gather(
        v, idx[:, None],
        lax.GatherDimensionNumbers(offset_dims=(), collapsed_slice_dims=(0,),
                                   start_index_map=(0,)),
        (1,), mode=lax.GatherScatterMode.PROMISE_IN_BOUNDS)


def _sc_body(src_ref, dst_ref, ts_hbm, td_hbm, mv_hbm, zr_hbm, out_hbm,
             sidx, didx, tsb, tdb, stage, mbuf, sem1, sem2, acc):
    c = lax.axis_index("c")
    s = lax.axis_index("s")
    pltpu.sync_copy(mv_hbm, mbuf)
    pltpu.sync_copy(zr_hbm, acc.at[pl.ds(s * ROWS_PT, ROWS_PT)])
    plsc.subcore_barrier()

    base = (c * 16 + s) * EPW
    iot = lax.iota(jnp.int32, 16)
    c01 = iot >> 3
    c23 = c01 + 2
    mv = mbuf[0]

    def chunk_body(i, carry):
        off = base + i * CHUNK
        pltpu.sync_copy(src_ref.at[pl.ds(off, CHUNK)], sidx)
        pltpu.sync_copy(dst_ref.at[pl.ds(off, CHUNK)], didx)
        pltpu.async_copy(ts_hbm.at[sidx], tsb, sem1).wait()
        pltpu.async_copy(td_hbm.at[didx], tdb, sem2).wait()
        for k in range(CHUNK):
            va = tsb[k, pl.ds(0, 16)]
            vd = tdb[k]
            z = va + vd
            alpha = jnp.maximum(z, 0.2 * z)
            ev = jnp.exp(alpha - mv)
            evm = jnp.where(off + k < E_TOT, ev, 0.0)
            w01s = tsb[k, pl.ds(16, 16)] * _dg(evm, c01)
            w23s = tsb[k, pl.ds(32, 16)] * _dg(evm, c23)
            stage[k, pl.ds(0, 16)] = w01s
            stage[k, pl.ds(16, 16)] = w23s
        pltpu.sync_copy(stage, acc.at[didx], add=True)
        return carry

    lax.fori_loop(0, NCHUNK, chunk_body, 0)
    plsc.subcore_barrier()
    pltpu.sync_copy(acc.at[pl.ds(s * ROWS_PT, ROWS_PT)],
                    out_hbm.at[c, pl.ds(s * ROWS_PT, ROWS_PT)])


def _sc_edge_pass(src, dst, tsrc, tdst, mvec, zrows):
    mesh = plsc.VectorSubcoreMesh(core_axis_name="c", subcore_axis_name="s")
    return pl.kernel(
        _sc_body,
        mesh=mesh,
        compiler_params=pltpu.CompilerParams(use_tc_tiling_on_sc=False),
        out_type=jax.ShapeDtypeStruct((2, NP, AW), jnp.float32),
        scratch_types=[
            pltpu.VMEM((CHUNK,), jnp.int32),
            pltpu.VMEM((CHUNK,), jnp.int32),
            pltpu.VMEM((CHUNK, TW), jnp.float32),
            pltpu.VMEM((CHUNK, 16), jnp.float32),
            pltpu.VMEM((CHUNK, AW), jnp.float32),
            pltpu.VMEM((1, 16), jnp.float32),
            pltpu.SemaphoreType.DMA,
            pltpu.SemaphoreType.DMA,
            pltpu.VMEM_SHARED((NP, AW), jnp.float32),
        ],
    )(src, dst, tsrc, tdst, mvec, zrows)


def _sc_e_body(src_ref, dst_ref, ta_hbm, td_hbm, mv_hbm, zr_hbm, out_hbm,
               sidx, didx, tab, tdb, stage, mbuf, sem1, sem2, acc):
    c = lax.axis_index("c")
    s = lax.axis_index("s")
    pltpu.sync_copy(mv_hbm, mbuf)
    pltpu.sync_copy(zr_hbm, acc.at[pl.ds(s * ROWS_PT, ROWS_PT)])
    plsc.subcore_barrier()

    base = (c * 16 + s) * EPW
    mv = mbuf[0]

    def chunk_body(i, carry):
        off = base + i * CHUNK
        pltpu.sync_copy(src_ref.at[pl.ds(off, CHUNK)], sidx)
        pltpu.sync_copy(dst_ref.at[pl.ds(off, CHUNK)], didx)
        pltpu.async_copy(ta_hbm.at[sidx], tab, sem1).wait()
        pltpu.async_copy(td_hbm.at[didx], tdb, sem2).wait()
        for k in range(CHUNK):
            z = tab[k] + tdb[k]
            alpha = jnp.maximum(z, 0.2 * z)
            ev = jnp.exp(alpha - mv)
            stage[k] = jnp.where(off + k < E_TOT, ev, 0.0)
        pltpu.sync_copy(stage, acc.at[didx], add=True)
        return carry

    lax.fori_loop(0, NCHUNK, chunk_body, 0)
    plsc.subcore_barrier()
    pltpu.sync_copy(acc.at[pl.ds(s * ROWS_PT, ROWS_PT)],
                    out_hbm.at[c, pl.ds(s * ROWS_PT, ROWS_PT)])


def _sc_e_pass(src, dst, tsa, tdst, mvec, zrows_e):
    mesh = plsc.VectorSubcoreMesh(core_axis_name="c", subcore_axis_name="s")
    return pl.kernel(
        _sc_e_body,
        mesh=mesh,
        compiler_params=pltpu.CompilerParams(use_tc_tiling_on_sc=False),
        out_type=jax.ShapeDtypeStruct((2, NP, EW), jnp.float32),
        scratch_types=[
            pltpu.VMEM((CHUNK,), jnp.int32),
            pltpu.VMEM((CHUNK,), jnp.int32),
            pltpu.VMEM((CHUNK, 16), jnp.float32),
            pltpu.VMEM((CHUNK, 16), jnp.float32),
            pltpu.VMEM((CHUNK, EW), jnp.float32),
            pltpu.VMEM((1, 16), jnp.float32),
            pltpu.SemaphoreType.DMA,
            pltpu.SemaphoreType.DMA,
            pltpu.VMEM_SHARED((NP, EW), jnp.float32),
        ],
    )(src, dst, tsa, tdst, mvec, zrows_e)


def _k2_body(acc_ref, acce_ref, h1_ref, wl_ref, bl_ref, bg_ref, wg_ref,
             asf_ref, adf_ref, S_ref, R_ref,
             h2_ref, ts_ref, td_ref, ta_ref, mx_ref):
    accs = acc_ref[0] + acc_ref[1]
    acce = acce_ref[0] + acce_ref[1]
    den = jnp.dot(acce[:, 0:4], R_ref[...],
                  preferred_element_type=jnp.float32) + 1e-16
    gat = accs / den + bg_ref[...]
    lin = jnp.dot(h1_ref[...], wl_ref[...],
                  preferred_element_type=jnp.float32) + bl_ref[...]
    h2 = _elu(gat + lin)
    h2_ref[...] = h2
    xw = jnp.dot(h2, wg_ref[...], preferred_element_type=jnp.float32)
    a_s = jnp.dot(xw * asf_ref[...], S_ref[...],
                  preferred_element_type=jnp.float32)
    a_d = jnp.dot(xw * adf_ref[...], S_ref[...],
                  preferred_element_type=jnp.float32)
    z = jnp.zeros((BN, 12), jnp.float32)
    ts_ref[...] = jnp.concatenate([a_s, z, xw], axis=1)
    td_ref[...] = jnp.concatenate([a_d, z], axis=1)
    ta_ref[...] = jnp.concatenate([a_s, z], axis=1)
    mx_ref[...] = jnp.concatenate(
        [jnp.max(a_s, axis=0), jnp.max(a_d, axis=0)], axis=0).reshape(1, 1, 8)


def _k2(acc, acce, h1, W_l1, b_l1, b_g1, W_g2, as_flat, ad_flat, S, R):
    return pl.pallas_call(
        _k2_body,
        grid=(GRID,),
        in_specs=[
            pl.BlockSpec((2, BN, AW), lambda i: (0, i, 0)),
            pl.BlockSpec((2, BN, EW), lambda i: (0, i, 0)),
            pl.BlockSpec((BN, HID), lambda i: (i, 0)),
            pl.BlockSpec((HID, HID), lambda i: (0, 0)),
            pl.BlockSpec((1, HID), lambda i: (0, 0)),
            pl.BlockSpec((1, HID), lambda i: (0, 0)),
            pl.BlockSpec((HID, HID), lambda i: (0, 0)),
            pl.BlockSpec((1, HID), lambda i: (0, 0)),
            pl.BlockSpec((1, HID), lambda i: (0, 0)),
            pl.BlockSpec((HID, H), lambda i: (0, 0)),
            pl.BlockSpec((H, HID), lambda i: (0, 0)),
        ],
        out_specs=[
            pl.BlockSpec((BN, HID), lambda i: (i, 0)),
            pl.BlockSpec((BN, TW), lambda i: (i, 0)),
            pl.BlockSpec((BN, 16), lambda i: (i, 0)),
            pl.BlockSpec((BN, 16), lambda i: (i, 0)),
            pl.BlockSpec((1, 1, 8), lambda i: (i, 0, 0)),
        ],
        out_shape=[
            jax.ShapeDtypeStruct((N, HID), jnp.float32),
            jax.ShapeDtypeStruct((N, TW), jnp.float32),
            jax.ShapeDtypeStruct((N, 16), jnp.float32),
            jax.ShapeDtypeStruct((N, 16), jnp.float32),
            jax.ShapeDtypeStruct((GRID, 1, 8), jnp.float32),
        ],
    )(acc, acce, h1, W_l1, b_l1.reshape(1, HID), b_g1.reshape(1, HID), W_g2,
      as_flat.reshape(1, HID), ad_flat.reshape(1, HID), S, R)


def _k3_body(acc_ref, acce_ref, h1_ref, h2_ref, wl_ref, bl_ref, bg_ref, R_ref,
             wfc_ref, bfc_ref, wout_ref, bout_ref, y_ref):
    accs = acc_ref[0] + acc_ref[1]
    acce = acce_ref[0] + acce_ref[1]
    den = jnp.dot(acce[:, 0:4], R_ref[...],
                  preferred_element_type=jnp.float32) + 1e-16
    gat = accs / den + bg_ref[...]
    lin = jnp.dot(h2_ref[...], wl_ref[...],
                  preferred_element_type=jnp.float32) + bl_ref[...]
    h3 = _elu(gat + lin)
    h = jnp.concatenate([h1_ref[...], h2_ref[...], h3], axis=1)
    hf = jnp.maximum(jnp.dot(h, wfc_ref[...],
                             preferred_element_type=jnp.float32)
                     + bfc_ref[...], 0.0)
    o = jnp.dot(hf, wout_ref[...],
                preferred_element_type=jnp.float32) + bout_ref[...]
    y_ref[...] = 1.0 / (1.0 + jnp.exp(-o))


def _k3(acc, acce, h1, h2, W_l2, b_l2, b_g2, R, W_fc, b_fc, W_out, b_out):
    return pl.pallas_call(
        _k3_body,
        grid=(GRID,),
        in_specs=[
            pl.BlockSpec((2, BN, AW), lambda i: (0, i, 0)),
            pl.BlockSpec((2, BN, EW), lambda i: (0, i, 0)),
            pl.BlockSpec((BN, HID), lambda i: (i, 0)),
            pl.BlockSpec((BN, HID), lambda i: (i, 0)),
            pl.BlockSpec((HID, HID), lambda i: (0, 0)),
            pl.BlockSpec((1, HID), lambda i: (0, 0)),
            pl.BlockSpec((1, HID), lambda i: (0, 0)),
            pl.BlockSpec((H, HID), lambda i: (0, 0)),
            pl.BlockSpec((3 * HID, 16), lambda i: (0, 0)),
            pl.BlockSpec((1, 16), lambda i: (0, 0)),
            pl.BlockSpec((16, 1), lambda i: (0, 0)),
            pl.BlockSpec((1, 1), lambda i: (0, 0)),
        ],
        out_specs=[pl.BlockSpec((BN, 1), lambda i: (i, 0))],
        out_shape=[jax.ShapeDtypeStruct((N, 1), jnp.float32)],
    )(acc, acce, h1, h2, W_l2, b_l2.reshape(1, HID), b_g2.reshape(1, HID), R,
      W_fc, b_fc.reshape(1, 16), W_out, b_out.reshape(1, 1))


_S_np = np.kron(np.eye(H, dtype=np.float32), np.ones((C, 1), np.float32))
_R_np = np.kron(np.eye(H, dtype=np.float32), np.ones((1, C), np.float32))


def _mvec(mx):
    m = jnp.max(mx, axis=(0, 1))
    Mh = jnp.maximum(m[0:4] + m[4:8], 0.0)
    return jnp.concatenate([Mh, jnp.full((12,), 1e30, jnp.float32)]).reshape(1, 16)


def kernel(x, edge_index, W_line, b_line, W_g1, a_src1, a_dst1, b_g1,
           W_l1, b_l1, W_g2, a_src2, a_dst2, b_g2, W_l2, b_l2,
           W_fc, b_fc, W_out, b_out):
    S = jnp.asarray(_S_np)
    R = jnp.asarray(_R_np)
    loop = jnp.arange(N, dtype=edge_index.dtype)
    padz = jnp.zeros((E_PAD - E_TOT,), edge_index.dtype)
    src = jnp.concatenate([edge_index[0], loop, padz])
    dst = jnp.concatenate([edge_index[1], loop, padz])
    zrows = jnp.zeros((ROWS_PT, AW), jnp.float32)
    zrows_e = jnp.zeros((ROWS_PT, EW), jnp.float32)

    h1, ts1, td1, ta1, mx1 = _k1(x, W_line, b_line, W_g1,
                                 a_src1.reshape(-1), a_dst1.reshape(-1), S)
    mv1 = _mvec(mx1)
    acc1 = _sc_edge_pass(src, dst, ts1, td1, mv1, zrows)
    ace1 = _sc_e_pass(src, dst, ta1, td1, mv1, zrows_e)
    h2, ts2, td2, ta2, mx2 = _k2(acc1, ace1, h1, W_l1, b_l1, b_g1, W_g2,
                                 a_src2.reshape(-1), a_dst2.reshape(-1), S, R)
    mv2 = _mvec(mx2)
    acc2 = _sc_edge_pass(src, dst, ts2, td2, mv2, zrows)
    ace2 = _sc_e_pass(src, dst, ta2, td2, mv2, zrows_e)
    y = _k3(acc2, ace2, h1, h2, W_l2, b_l2, b_g2, R, W_fc, b_fc, W_out, b_out)
    return y[0].reshape(N)


# speedup vs baseline: 92.2321x; 1.2791x over previous
import functools
import numpy as np
import jax
import jax.numpy as jnp
from jax import lax
from jax.experimental import pallas as pl
from jax.experimental.pallas import tpu as pltpu
from jax.experimental.pallas import tpu_sc as plsc

N = 50000
E = 1600000
D_IN = 1346
H = 4
C = 8
HID = 32

BN = 1000
GRID = N // BN
E_TOT = E + N
NW = 32
CHUNK = 256
EPW = 51712
E_PAD = NW * EPW
NCHUNK = EPW // CHUNK
NP = 50048
ROWS_PT = NP // 16
TW = 48
AW = 32
EW = 16


def _leaky(x):
    return jnp.where(x >= 0, x, 0.2 * x)


def _elu(x):
    return jnp.where(x > 0, x, jnp.exp(jnp.minimum(x, 0.0)) - 1.0)


def _k1_body(x_ref, wl_ref, bl_ref, wg_ref, asf_ref, adf_ref, S_ref,
             h1_ref, ts_ref, td_ref, ta_ref, mx_ref):
    h1 = jnp.maximum(jnp.dot(x_ref[...], wl_ref[...],
                             preferred_element_type=jnp.float32)
                     + bl_ref[...], 0.0)
    h1_ref[...] = h1
    xw = jnp.dot(h1, wg_ref[...], preferred_element_type=jnp.float32)
    a_s = jnp.dot(xw * asf_ref[...], S_ref[...],
                  preferred_element_type=jnp.float32)
    a_d = jnp.dot(xw * adf_ref[...], S_ref[...],
                  preferred_element_type=jnp.float32)
    z = jnp.zeros((BN, 12), jnp.float32)
    ts_ref[...] = jnp.concatenate([a_s, z, xw], axis=1)
    td_ref[...] = jnp.concatenate([a_d, z], axis=1)
    ta_ref[...] = jnp.concatenate([a_s, z], axis=1)
    mx_ref[...] = jnp.concatenate(
        [jnp.max(a_s, axis=0), jnp.max(a_d, axis=0)], axis=0).reshape(1, 1, 8)


def _k1(x, W_line, b_line, W_g1, as_flat, ad_flat, S):
    return pl.pallas_call(
        _k1_body,
        grid=(GRID,),
        in_specs=[
            pl.BlockSpec((BN, D_IN), lambda i: (i, 0)),
            pl.BlockSpec((D_IN, HID), lambda i: (0, 0)),
            pl.BlockSpec((1, HID), lambda i: (0, 0)),
            pl.BlockSpec((HID, HID), lambda i: (0, 0)),
            pl.BlockSpec((1, HID), lambda i: (0, 0)),
            pl.BlockSpec((1, HID), lambda i: (0, 0)),
            pl.BlockSpec((HID, H), lambda i: (0, 0)),
        ],
        out_specs=[
            pl.BlockSpec((BN, HID), lambda i: (i, 0)),
            pl.BlockSpec((BN, TW), lambda i: (i, 0)),
            pl.BlockSpec((BN, 16), lambda i: (i, 0)),
            pl.BlockSpec((BN, 16), lambda i: (i, 0)),
            pl.BlockSpec((1, 1, 8), lambda i: (i, 0, 0)),
        ],
        out_shape=[
            jax.ShapeDtypeStruct((N, HID), jnp.float32),
            jax.ShapeDtypeStruct((N, TW), jnp.float32),
            jax.ShapeDtypeStruct((N, 16), jnp.float32),
            jax.ShapeDtypeStruct((N, 16), jnp.float32),
            jax.ShapeDtypeStruct((GRID, 1, 8), jnp.float32),
        ],
    )(x, W_line, b_line.reshape(1, HID), W_g1,
      as_flat.reshape(1, HID), ad_flat.reshape(1, HID), S)


def _dg(v, idx):
    return lax.gather(
        v, idx[:, None],
        lax.GatherDimensionNumbers(offset_dims=(), collapsed_slice_dims=(0,),
                                   start_index_map=(0,)),
        (1,), mode=lax.GatherScatterMode.PROMISE_IN_BOUNDS)


def _sc_body(src_ref, dst_ref, ts_hbm, td_hbm, mv_hbm, zr_hbm, out_hbm,
             sidx, didx, tsb, tdb, stage, mbuf, sem1, sem2, acc):
    c = lax.axis_index("c")
    s = lax.axis_index("s")
    pltpu.sync_copy(mv_hbm, mbuf)
    pltpu.sync_copy(zr_hbm, acc.at[pl.ds(s * ROWS_PT, ROWS_PT)])
    plsc.subcore_barrier()

    base = (c * 16 + s) * EPW
    iot = lax.iota(jnp.int32, 16)
    c01 = iot >> 3
    c23 = c01 + 2
    mv = mbuf[0]

    def chunk_body(i, carry):
        off = base + i * CHUNK
        pltpu.sync_copy(src_ref.at[pl.ds(off, CHUNK)], sidx)
        pltpu.sync_copy(dst_ref.at[pl.ds(off, CHUNK)], didx)
        pltpu.async_copy(ts_hbm.at[sidx], tsb, sem1).wait()
        pltpu.async_copy(td_hbm.at[didx], tdb, sem2).wait()
        for k in range(CHUNK):
            va = tsb[k, pl.ds(0, 16)]
            vd = tdb[k]
            z = va + vd
            alpha = jnp.maximum(z, 0.2 * z)
            ev = jnp.exp(alpha - mv)
            evm = jnp.where(off + k < E_TOT, ev, 0.0)
            w01s = tsb[k, pl.ds(16, 16)] * _dg(evm, c01)
            w23s = tsb[k, pl.ds(32, 16)] * _dg(evm, c23)
            stage[k, pl.ds(0, 16)] = w01s
            stage[k, pl.ds(16, 16)] = w23s
        pltpu.sync_copy(stage, acc.at[didx], add=True)
        return carry

    lax.fori_loop(0, NCHUNK, chunk_body, 0)
    plsc.subcore_barrier()
    pltpu.sync_copy(acc.at[pl.ds(s * ROWS_PT, ROWS_PT)],
                    out_hbm.at[c, pl.ds(s * ROWS_PT, ROWS_PT)])


def _sc_edge_pass(src, dst, tsrc, tdst, mvec, zrows):
    mesh = plsc.VectorSubcoreMesh(core_axis_name="c", subcore_axis_name="s")
    return pl.kernel(
        _sc_body,
        mesh=mesh,
        compiler_params=pltpu.CompilerParams(use_tc_tiling_on_sc=False),
        out_type=jax.ShapeDtypeStruct((2, NP, AW), jnp.float32),
        scratch_types=[
            pltpu.VMEM((CHUNK,), jnp.int32),
            pltpu.VMEM((CHUNK,), jnp.int32),
            pltpu.VMEM((CHUNK, TW), jnp.float32),
            pltpu.VMEM((CHUNK, 16), jnp.float32),
            pltpu.VMEM((CHUNK, AW), jnp.float32),
            pltpu.VMEM((1, 16), jnp.float32),
            pltpu.SemaphoreType.DMA,
            pltpu.SemaphoreType.DMA,
            pltpu.VMEM_SHARED((NP, AW), jnp.float32),
        ],
    )(src, dst, tsrc, tdst, mvec, zrows)


def _sc_e_body(src_ref, dst_ref, ta_hbm, td_hbm, mv_hbm, zr_hbm, out_hbm,
               sidx, didx, tab, tdb, stage, mbuf, sem1, sem2, acc):
    c = lax.axis_index("c")
    s = lax.axis_index("s")
    pltpu.sync_copy(mv_hbm, mbuf)
    pltpu.sync_copy(zr_hbm, acc.at[pl.ds(s * ROWS_PT, ROWS_PT)])
    plsc.subcore_barrier()

    base = (c * 16 + s) * EPW
    mv = mbuf[0]

    def chunk_body(i, carry):
        off = base + i * CHUNK
        pltpu.sync_copy(src_ref.at[pl.ds(off, CHUNK)], sidx)
        pltpu.sync_copy(dst_ref.at[pl.ds(off, CHUNK)], didx)
        pltpu.async_copy(ta_hbm.at[sidx], tab, sem1).wait()
        pltpu.async_copy(td_hbm.at[didx], tdb, sem2).wait()
        for k in range(CHUNK):
            z = tab[k] + tdb[k]
            alpha = jnp.maximum(z, 0.2 * z)
            ev = jnp.exp(alpha - mv)
            stage[k] = jnp.where(off + k < E_TOT, ev, 0.0)
        pltpu.sync_copy(stage, acc.at[didx], add=True)
        return carry

    lax.fori_loop(0, NCHUNK, chunk_body, 0)
    plsc.subcore_barrier()
    pltpu.sync_copy(acc.at[pl.ds(s * ROWS_PT, ROWS_PT)],
                    out_hbm.at[c, pl.ds(s * ROWS_PT, ROWS_PT)])


def _sc_e_pass(src, dst, tsa, tdst, mvec, zrows_e):
    mesh = plsc.VectorSubcoreMesh(core_axis_name="c", subcore_axis_name="s")
    return pl.kernel(
        _sc_e_body,
        mesh=mesh,
        compiler_params=pltpu.CompilerParams(use_tc_tiling_on_sc=False),
        out_type=jax.ShapeDtypeStruct((2, NP, EW), jnp.float32),
        scratch_types=[
            pltpu.VMEM((CHUNK,), jnp.int32),
            pltpu.VMEM((CHUNK,), jnp.int32),
            pltpu.VMEM((CHUNK, 16), jnp.float32),
            pltpu.VMEM((CHUNK, 16), jnp.float32),
            pltpu.VMEM((CHUNK, EW), jnp.float32),
            pltpu.VMEM((1, 16), jnp.float32),
            pltpu.SemaphoreType.DMA,
            pltpu.SemaphoreType.DMA,
            pltpu.VMEM_SHARED((NP, EW), jnp.float32),
        ],
    )(src, dst, tsa, tdst, mvec, zrows_e)


def _k2_body(acc_ref, acce_ref, h1_ref, wl_ref, bl_ref, bg_ref, wg_ref,
             asf_ref, adf_ref, S_ref, R_ref,
             h2_ref, ts_ref, td_ref, ta_ref, mx_ref):
    accs = acc_ref[0] + acc_ref[1]
    acce = acce_ref[0] + acce_ref[1]
    den = jnp.dot(acce[:, 0:4], R_ref[...],
                  preferred_element_type=jnp.float32) + 1e-16
    gat = accs / den + bg_ref[...]
    lin = jnp.dot(h1_ref[...], wl_ref[...],
                  preferred_element_type=jnp.float32) + bl_ref[...]
    h2 = _elu(gat + lin)
    h2_ref[...] = h2
    xw = jnp.dot(h2, wg_ref[...], preferred_element_type=jnp.float32)
    a_s = jnp.dot(xw * asf_ref[...], S_ref[...],
                  preferred_element_type=jnp.float32)
    a_d = jnp.dot(xw * adf_ref[...], S_ref[...],
                  preferred_element_type=jnp.float32)
    z = jnp.zeros((BN, 12), jnp.float32)
    ts_ref[...] = jnp.concatenate([a_s, z, xw], axis=1)
    td_ref[...] = jnp.concatenate([a_d, z], axis=1)
    ta_ref[...] = jnp.concatenate([a_s, z], axis=1)
    mx_ref[...] = jnp.concatenate(
        [jnp.max(a_s, axis=0), jnp.max(a_d, axis=0)], axis=0).reshape(1, 1, 8)


def _k2(acc, acce, h1, W_l1, b_l1, b_g1, W_g2, as_flat, ad_flat, S, R):
    return pl.pallas_call(
        _k2_body,
        grid=(GRID,),
        in_specs=[
            pl.BlockSpec((2, BN, AW), lambda i: (0, i, 0)),
            pl.BlockSpec((2, BN, EW), lambda i: (0, i, 0)),
            pl.BlockSpec((BN, HID), lambda i: (i, 0)),
            pl.BlockSpec((HID, HID), lambda i: (0, 0)),
            pl.BlockSpec((1, HID), lambda i: (0, 0)),
            pl.BlockSpec((1, HID), lambda i: (0, 0)),
            pl.BlockSpec((HID, HID), lambda i: (0, 0)),
            pl.BlockSpec((1, HID), lambda i: (0, 0)),
            pl.BlockSpec((1, HID), lambda i: (0, 0)),
            pl.BlockSpec((HID, H), lambda i: (0, 0)),
            pl.BlockSpec((H, HID), lambda i: (0, 0)),
        ],
        out_specs=[
            pl.BlockSpec((BN, HID), lambda i: (i, 0)),
            pl.BlockSpec((BN, TW), lambda i: (i, 0)),
            pl.BlockSpec((BN, 16), lambda i: (i, 0)),
            pl.BlockSpec((BN, 16), lambda i: (i, 0)),
            pl.BlockSpec((1, 1, 8), lambda i: (i, 0, 0)),
        ],
        out_shape=[
            jax.ShapeDtypeStruct((N, HID), jnp.float32),
            jax.ShapeDtypeStruct((N, TW), jnp.float32),
            jax.ShapeDtypeStruct((N, 16), jnp.float32),
            jax.ShapeDtypeStruct((N, 16), jnp.float32),
            jax.ShapeDtypeStruct((GRID, 1, 8), jnp.float32),
        ],
    )(acc, acce, h1, W_l1, b_l1.reshape(1, HID), b_g1.reshape(1, HID), W_g2,
      as_flat.reshape(1, HID), ad_flat.reshape(1, HID), S, R)


def _k3_body(acc_ref, acce_ref, h1_ref, h2_ref, wl_ref, bl_ref, bg_ref, R_ref,
             wfc_ref, bfc_ref, wout_ref, bout_ref, y_ref):
    accs = acc_ref[0] + acc_ref[1]
    acce = acce_ref[0] + acce_ref[1]
    den = jnp.dot(acce[:, 0:4], R_ref[...],
                  preferred_element_type=jnp.float32) + 1e-16
    gat = accs / den + bg_ref[...]
    lin = jnp.dot(h2_ref[...], wl_ref[...],
                  preferred_element_type=jnp.float32) + bl_ref[...]
    h3 = _elu(gat + lin)
    h = jnp.concatenate([h1_ref[...], h2_ref[...], h3], axis=1)
    hf = jnp.maximum(jnp.dot(h, wfc_ref[...],
                             preferred_element_type=jnp.float32)
                     + bfc_ref[...], 0.0)
    o = jnp.dot(hf, wout_ref[...],
                preferred_element_type=jnp.float32) + bout_ref[...]
    y_ref[...] = 1.0 / (1.0 + jnp.exp(-o))


def _k3(acc, acce, h1, h2, W_l2, b_l2, b_g2, R, W_fc, b_fc, W_out, b_out):
    return pl.pallas_call(
        _k3_body,
        grid=(GRID,),
        in_specs=[
            pl.BlockSpec((2, BN, AW), lambda i: (0, i, 0)),
            pl.BlockSpec((2, BN, EW), lambda i: (0, i, 0)),
            pl.BlockSpec((BN, HID), lambda i: (i, 0)),
            pl.BlockSpec((BN, HID), lambda i: (i, 0)),
            pl.BlockSpec((HID, HID), lambda i: (0, 0)),
            pl.BlockSpec((1, HID), lambda i: (0, 0)),
            pl.BlockSpec((1, HID), lambda i: (0, 0)),
            pl.BlockSpec((H, HID), lambda i: (0, 0)),
            pl.BlockSpec((3 * HID, 16), lambda i: (0, 0)),
            pl.BlockSpec((1, 16), lambda i: (0, 0)),
            pl.BlockSpec((16, 1), lambda i: (0, 0)),
            pl.BlockSpec((1, 1), lambda i: (0, 0)),
        ],
        out_specs=[pl.BlockSpec((BN, 1), lambda i: (i, 0))],
        out_shape=[jax.ShapeDtypeStruct((N, 1), jnp.float32)],
    )(acc, acce, h1, h2, W_l2, b_l2.reshape(1, HID), b_g2.reshape(1, HID), R,
      W_fc, b_fc.reshape(1, 16), W_out, b_out.reshape(1, 1))


_S_np = np.kron(np.eye(H, dtype=np.float32), np.ones((C, 1), np.float32))
_R_np = np.kron(np.eye(H, dtype=np.float32), np.ones((1, C), np.float32))


def _mvec(mx):
    m = jnp.max(mx, axis=(0, 1))
    Mh = jnp.maximum(m[0:4] + m[4:8], 0.0)
    return jnp.concatenate([Mh, jnp.full((12,), 1e30, jnp.float32)]).reshape(1, 16)


def kernel(x, edge_index, W_line, b_line, W_g1, a_src1, a_dst1, b_g1,
           W_l1, b_l1, W_g2, a_src2, a_dst2, b_g2, W_l2, b_l2,
           W_fc, b_fc, W_out, b_out):
    S = jnp.asarray(_S_np)
    R = jnp.asarray(_R_np)
    loop = jnp.arange(N, dtype=edge_index.dtype)
    padz = jnp.zeros((E_PAD - E_TOT,), edge_index.dtype)
    src = jnp.concatenate([edge_index[0], loop, padz])
    dst = jnp.concatenate([edge_index[1], loop, padz])
    zrows = jnp.zeros((ROWS_PT, AW), jnp.float32)
    zrows_e = jnp.zeros((ROWS_PT, EW), jnp.float32)

    h1, ts1, td1, ta1, mx1 = _k1(x, W_line, b_line, W_g1,
                                 a_src1.reshape(-1), a_dst1.reshape(-1), S)
    mv1 = _mvec(mx1)
    acc1 = _sc_edge_pass(src, dst, ts1, td1, mv1, zrows)
    ace1 = _sc_e_pass(src, dst, ta1, td1, mv1, zrows_e)
    h2, ts2, td2, ta2, mx2 = _k2(acc1, ace1, h1, W_l1, b_l1, b_g1, W_g2,
                                 a_src2.reshape(-1), a_dst2.reshape(-1), S, R)
    mv2 = _mvec(mx2)
    acc2 = _sc_edge_pass(src, dst, ts2, td2, mv2, zrows)
    ace2 = _sc_e_pass(src, dst, ta2, td2, mv2, zrows_e)
    y = _k3(acc2, ace2, h1, h2, W_l2, b_l2, b_g2, R, W_fc, b_fc, W_out, b_out)
    return y[0].reshape(N)


# speedup vs baseline: 106.9570x; 1.1597x over previous
import functools
import numpy as np
import jax
import jax.numpy as jnp
from jax import lax
from jax.experimental import pallas as pl
from jax.experimental.pallas import tpu as pltpu
from jax.experimental.pallas import tpu_sc as plsc

N = 50000
E = 1600000
D_IN = 1346
H = 4
C = 8
HID = 32

BN = 1000
GRID = N // BN
E_TOT = E + N
NW = 32
CHUNK = 256
EPW = 51712
E_PAD = NW * EPW
NCHUNK = EPW // CHUNK
NP = 50048
ROWS_PT = NP // 16
TW = 48
AW = 32
EW = 16


def _leaky(x):
    return jnp.where(x >= 0, x, 0.2 * x)


def _elu(x):
    return jnp.where(x > 0, x, jnp.exp(jnp.minimum(x, 0.0)) - 1.0)


def _k1_body(x_ref, wl_ref, bl_ref, wg_ref, asf_ref, adf_ref, S_ref,
             h1_ref, ts_ref, td_ref, ta_ref, mx_ref):
    h1 = jnp.maximum(jnp.dot(x_ref[...], wl_ref[...],
                             preferred_element_type=jnp.float32)
                     + bl_ref[...], 0.0)
    h1_ref[...] = h1
    xw = jnp.dot(h1, wg_ref[...], preferred_element_type=jnp.float32)
    a_s = jnp.dot(xw * asf_ref[...], S_ref[...],
                  preferred_element_type=jnp.float32)
    a_d = jnp.dot(xw * adf_ref[...], S_ref[...],
                  preferred_element_type=jnp.float32)
    z = jnp.zeros((BN, 12), jnp.float32)
    ts_ref[...] = jnp.concatenate([a_s, z, xw], axis=1)
    td_ref[...] = jnp.concatenate([a_d, z], axis=1)
    ta_ref[...] = jnp.concatenate([a_s, z], axis=1)
    mx_ref[...] = jnp.concatenate(
        [jnp.max(a_s, axis=0), jnp.max(a_d, axis=0)], axis=0).reshape(1, 1, 8)


def _k1(x, W_line, b_line, W_g1, as_flat, ad_flat, S):
    return pl.pallas_call(
        _k1_body,
        grid=(GRID,),
        in_specs=[
            pl.BlockSpec((BN, D_IN), lambda i: (i, 0)),
            pl.BlockSpec((D_IN, HID), lambda i: (0, 0)),
            pl.BlockSpec((1, HID), lambda i: (0, 0)),
            pl.BlockSpec((HID, HID), lambda i: (0, 0)),
            pl.BlockSpec((1, HID), lambda i: (0, 0)),
            pl.BlockSpec((1, HID), lambda i: (0, 0)),
            pl.BlockSpec((HID, H), lambda i: (0, 0)),
        ],
        out_specs=[
            pl.BlockSpec((BN, HID), lambda i: (i, 0)),
            pl.BlockSpec((BN, TW), lambda i: (i, 0)),
            pl.BlockSpec((BN, 16), lambda i: (i, 0)),
            pl.BlockSpec((BN, 16), lambda i: (i, 0)),
            pl.BlockSpec((1, 1, 8), lambda i: (i, 0, 0)),
        ],
        out_shape=[
            jax.ShapeDtypeStruct((N, HID), jnp.float32),
            jax.ShapeDtypeStruct((N, TW), jnp.float32),
            jax.ShapeDtypeStruct((N, 16), jnp.float32),
            jax.ShapeDtypeStruct((N, 16), jnp.float32),
            jax.ShapeDtypeStruct((GRID, 1, 8), jnp.float32),
        ],
    )(x, W_line, b_line.reshape(1, HID), W_g1,
      as_flat.reshape(1, HID), ad_flat.reshape(1, HID), S)


def _dg(v, idx):
    return lax.gather(
        v, idx[:, None],
        lax.GatherDimensionNumbers(offset_dims=(), collapsed_slice_dims=(0,),
                                   start_index_map=(0,)),
        (1,), mode=lax.GatherScatterMode.PROMISE_IN_BOUNDS)


def _sc_body(src_ref, dst_ref, ts_hbm, td_hbm, mv_hbm, zr_hbm, out_hbm,
             sidx, didx, tsb, tdb, stage, mbuf, sem1, sem2, acc):
    c = lax.axis_index("c")
    s = lax.axis_index("s")
    pltpu.sync_copy(mv_hbm, mbuf)
    pltpu.sync_copy(zr_hbm, acc.at[pl.ds(s * ROWS_PT, ROWS_PT)])
    plsc.subcore_barrier()

    base = (c * 16 + s) * EPW
    iot = lax.iota(jnp.int32, 16)
    c01 = iot >> 3
    c23 = c01 + 2
    mv = mbuf[0]

    def chunk_body(i, carry):
        off = base + i * CHUNK
        pltpu.sync_copy(src_ref.at[pl.ds(off, CHUNK)], sidx)
        pltpu.sync_copy(dst_ref.at[pl.ds(off, CHUNK)], didx)
        cp1 = pltpu.async_copy(ts_hbm.at[sidx], tsb, sem1)
        cp2 = pltpu.async_copy(td_hbm.at[didx], tdb, sem2)
        cp1.wait()
        cp2.wait()
        for k in range(CHUNK):
            va = tsb[k, pl.ds(0, 16)]
            vd = tdb[k]
            z = va + vd
            alpha = jnp.maximum(z, 0.2 * z)
            ev = jnp.exp(alpha - mv)
            evm = jnp.where(off + k < E_TOT, ev, 0.0)
            w01s = tsb[k, pl.ds(16, 16)] * _dg(evm, c01)
            w23s = tsb[k, pl.ds(32, 16)] * _dg(evm, c23)
            stage[k, pl.ds(0, 16)] = w01s
            stage[k, pl.ds(16, 16)] = w23s
        pltpu.sync_copy(stage, acc.at[didx], add=True)
        return carry

    lax.fori_loop(0, NCHUNK, chunk_body, 0)
    plsc.subcore_barrier()
    pltpu.sync_copy(acc.at[pl.ds(s * ROWS_PT, ROWS_PT)],
                    out_hbm.at[c, pl.ds(s * ROWS_PT, ROWS_PT)])


def _sc_edge_pass(src, dst, tsrc, tdst, mvec, zrows):
    mesh = plsc.VectorSubcoreMesh(core_axis_name="c", subcore_axis_name="s")
    return pl.kernel(
        _sc_body,
        mesh=mesh,
        compiler_params=pltpu.CompilerParams(use_tc_tiling_on_sc=False),
        out_type=jax.ShapeDtypeStruct((2, NP, AW), jnp.float32),
        scratch_types=[
            pltpu.VMEM((CHUNK,), jnp.int32),
            pltpu.VMEM((CHUNK,), jnp.int32),
            pltpu.VMEM((CHUNK, TW), jnp.float32),
            pltpu.VMEM((CHUNK, 16), jnp.float32),
            pltpu.VMEM((CHUNK, AW), jnp.float32),
            pltpu.VMEM((1, 16), jnp.float32),
            pltpu.SemaphoreType.DMA,
            pltpu.SemaphoreType.DMA,
            pltpu.VMEM_SHARED((NP, AW), jnp.float32),
        ],
    )(src, dst, tsrc, tdst, mvec, zrows)


def _sc_e_body(src_ref, dst_ref, ta_hbm, td_hbm, mv_hbm, zr_hbm, out_hbm,
               sidx, didx, tab, tdb, stage, mbuf, sem1, sem2, acc):
    c = lax.axis_index("c")
    s = lax.axis_index("s")
    pltpu.sync_copy(mv_hbm, mbuf)
    pltpu.sync_copy(zr_hbm, acc.at[pl.ds(s * ROWS_PT, ROWS_PT)])
    plsc.subcore_barrier()

    base = (c * 16 + s) * EPW
    mv = mbuf[0]

    def chunk_body(i, carry):
        off = base + i * CHUNK
        pltpu.sync_copy(src_ref.at[pl.ds(off, CHUNK)], sidx)
        pltpu.sync_copy(dst_ref.at[pl.ds(off, CHUNK)], didx)
        cp1 = pltpu.async_copy(ta_hbm.at[sidx], tab, sem1)
        cp2 = pltpu.async_copy(td_hbm.at[didx], tdb, sem2)
        cp1.wait()
        cp2.wait()
        for k in range(CHUNK):
            z = tab[k] + tdb[k]
            alpha = jnp.maximum(z, 0.2 * z)
            ev = jnp.exp(alpha - mv)
            stage[k] = jnp.where(off + k < E_TOT, ev, 0.0)
        pltpu.sync_copy(stage, acc.at[didx], add=True)
        return carry

    lax.fori_loop(0, NCHUNK, chunk_body, 0)
    plsc.subcore_barrier()
    pltpu.sync_copy(acc.at[pl.ds(s * ROWS_PT, ROWS_PT)],
                    out_hbm.at[c, pl.ds(s * ROWS_PT, ROWS_PT)])


def _sc_e_pass(src, dst, tsa, tdst, mvec, zrows_e):
    mesh = plsc.VectorSubcoreMesh(core_axis_name="c", subcore_axis_name="s")
    return pl.kernel(
        _sc_e_body,
        mesh=mesh,
        compiler_params=pltpu.CompilerParams(use_tc_tiling_on_sc=False),
        out_type=jax.ShapeDtypeStruct((2, NP, EW), jnp.float32),
        scratch_types=[
            pltpu.VMEM((CHUNK,), jnp.int32),
            pltpu.VMEM((CHUNK,), jnp.int32),
            pltpu.VMEM((CHUNK, 16), jnp.float32),
            pltpu.VMEM((CHUNK, 16), jnp.float32),
            pltpu.VMEM((CHUNK, EW), jnp.float32),
            pltpu.VMEM((1, 16), jnp.float32),
            pltpu.SemaphoreType.DMA,
            pltpu.SemaphoreType.DMA,
            pltpu.VMEM_SHARED((NP, EW), jnp.float32),
        ],
    )(src, dst, tsa, tdst, mvec, zrows_e)


def _k2_body(acc_ref, acce_ref, h1_ref, wl_ref, bl_ref, bg_ref, wg_ref,
             asf_ref, adf_ref, S_ref, R_ref,
             h2_ref, ts_ref, td_ref, ta_ref, mx_ref):
    accs = acc_ref[0] + acc_ref[1]
    acce = acce_ref[0] + acce_ref[1]
    den = jnp.dot(acce[:, 0:4], R_ref[...],
                  preferred_element_type=jnp.float32) + 1e-16
    gat = accs / den + bg_ref[...]
    lin = jnp.dot(h1_ref[...], wl_ref[...],
                  preferred_element_type=jnp.float32) + bl_ref[...]
    h2 = _elu(gat + lin)
    h2_ref[...] = h2
    xw = jnp.dot(h2, wg_ref[...], preferred_element_type=jnp.float32)
    a_s = jnp.dot(xw * asf_ref[...], S_ref[...],
                  preferred_element_type=jnp.float32)
    a_d = jnp.dot(xw * adf_ref[...], S_ref[...],
                  preferred_element_type=jnp.float32)
    z = jnp.zeros((BN, 12), jnp.float32)
    ts_ref[...] = jnp.concatenate([a_s, z, xw], axis=1)
    td_ref[...] = jnp.concatenate([a_d, z], axis=1)
    ta_ref[...] = jnp.concatenate([a_s, z], axis=1)
    mx_ref[...] = jnp.concatenate(
        [jnp.max(a_s, axis=0), jnp.max(a_d, axis=0)], axis=0).reshape(1, 1, 8)


def _k2(acc, acce, h1, W_l1, b_l1, b_g1, W_g2, as_flat, ad_flat, S, R):
    return pl.pallas_call(
        _k2_body,
        grid=(GRID,),
        in_specs=[
            pl.BlockSpec((2, BN, AW), lambda i: (0, i, 0)),
            pl.BlockSpec((2, BN, EW), lambda i: (0, i, 0)),
            pl.BlockSpec((BN, HID), lambda i: (i, 0)),
            pl.BlockSpec((HID, HID), lambda i: (0, 0)),
            pl.BlockSpec((1, HID), lambda i: (0, 0)),
            pl.BlockSpec((1, HID), lambda i: (0, 0)),
            pl.BlockSpec((HID, HID), lambda i: (0, 0)),
            pl.BlockSpec((1, HID), lambda i: (0, 0)),
            pl.BlockSpec((1, HID), lambda i: (0, 0)),
            pl.BlockSpec((HID, H), lambda i: (0, 0)),
            pl.BlockSpec((H, HID), lambda i: (0, 0)),
        ],
        out_specs=[
            pl.BlockSpec((BN, HID), lambda i: (i, 0)),
            pl.BlockSpec((BN, TW), lambda i: (i, 0)),
            pl.BlockSpec((BN, 16), lambda i: (i, 0)),
            pl.BlockSpec((BN, 16), lambda i: (i, 0)),
            pl.BlockSpec((1, 1, 8), lambda i: (i, 0, 0)),
        ],
        out_shape=[
            jax.ShapeDtypeStruct((N, HID), jnp.float32),
            jax.ShapeDtypeStruct((N, TW), jnp.float32),
            jax.ShapeDtypeStruct((N, 16), jnp.float32),
            jax.ShapeDtypeStruct((N, 16), jnp.float32),
            jax.ShapeDtypeStruct((GRID, 1, 8), jnp.float32),
        ],
    )(acc, acce, h1, W_l1, b_l1.reshape(1, HID), b_g1.reshape(1, HID), W_g2,
      as_flat.reshape(1, HID), ad_flat.reshape(1, HID), S, R)


def _k3_body(acc_ref, acce_ref, h1_ref, h2_ref, wl_ref, bl_ref, bg_ref, R_ref,
             wfc_ref, bfc_ref, wout_ref, bout_ref, y_ref):
    accs = acc_ref[0] + acc_ref[1]
    acce = acce_ref[0] + acce_ref[1]
    den = jnp.dot(acce[:, 0:4], R_ref[...],
                  preferred_element_type=jnp.float32) + 1e-16
    gat = accs / den + bg_ref[...]
    lin = jnp.dot(h2_ref[...], wl_ref[...],
                  preferred_element_type=jnp.float32) + bl_ref[...]
    h3 = _elu(gat + lin)
    h = jnp.concatenate([h1_ref[...], h2_ref[...], h3], axis=1)
    hf = jnp.maximum(jnp.dot(h, wfc_ref[...],
                             preferred_element_type=jnp.float32)
                     + bfc_ref[...], 0.0)
    o = jnp.dot(hf, wout_ref[...],
                preferred_element_type=jnp.float32) + bout_ref[...]
    y_ref[...] = 1.0 / (1.0 + jnp.exp(-o))


def _k3(acc, acce, h1, h2, W_l2, b_l2, b_g2, R, W_fc, b_fc, W_out, b_out):
    return pl.pallas_call(
        _k3_body,
        grid=(GRID,),
        in_specs=[
            pl.BlockSpec((2, BN, AW), lambda i: (0, i, 0)),
            pl.BlockSpec((2, BN, EW), lambda i: (0, i, 0)),
            pl.BlockSpec((BN, HID), lambda i: (i, 0)),
            pl.BlockSpec((BN, HID), lambda i: (i, 0)),
            pl.BlockSpec((HID, HID), lambda i: (0, 0)),
            pl.BlockSpec((1, HID), lambda i: (0, 0)),
            pl.BlockSpec((1, HID), lambda i: (0, 0)),
            pl.BlockSpec((H, HID), lambda i: (0, 0)),
            pl.BlockSpec((3 * HID, 16), lambda i: (0, 0)),
            pl.BlockSpec((1, 16), lambda i: (0, 0)),
            pl.BlockSpec((16, 1), lambda i: (0, 0)),
            pl.BlockSpec((1, 1), lambda i: (0, 0)),
        ],
        out_specs=[pl.BlockSpec((BN, 1), lambda i: (i, 0))],
        out_shape=[jax.ShapeDtypeStruct((N, 1), jnp.float32)],
    )(acc, acce, h1, h2, W_l2, b_l2.reshape(1, HID), b_g2.reshape(1, HID), R,
      W_fc, b_fc.reshape(1, 16), W_out, b_out.reshape(1, 1))


_S_np = np.kron(np.eye(H, dtype=np.float32), np.ones((C, 1), np.float32))
_R_np = np.kron(np.eye(H, dtype=np.float32), np.ones((1, C), np.float32))


def _mvec(mx):
    m = jnp.max(mx, axis=(0, 1))
    Mh = jnp.maximum(m[0:4] + m[4:8], 0.0)
    return jnp.concatenate([Mh, jnp.full((12,), 1e30, jnp.float32)]).reshape(1, 16)


def kernel(x, edge_index, W_line, b_line, W_g1, a_src1, a_dst1, b_g1,
           W_l1, b_l1, W_g2, a_src2, a_dst2, b_g2, W_l2, b_l2,
           W_fc, b_fc, W_out, b_out):
    S = jnp.asarray(_S_np)
    R = jnp.asarray(_R_np)
    loop = jnp.arange(N, dtype=edge_index.dtype)
    padz = jnp.zeros((E_PAD - E_TOT,), edge_index.dtype)
    src = jnp.concatenate([edge_index[0], loop, padz])
    dst = jnp.concatenate([edge_index[1], loop, padz])
    zrows = jnp.zeros((ROWS_PT, AW), jnp.float32)
    zrows_e = jnp.zeros((ROWS_PT, EW), jnp.float32)

    h1, ts1, td1, ta1, mx1 = _k1(x, W_line, b_line, W_g1,
                                 a_src1.reshape(-1), a_dst1.reshape(-1), S)
    mv1 = _mvec(mx1)
    acc1 = _sc_edge_pass(src, dst, ts1, td1, mv1, zrows)
    ace1 = _sc_e_pass(src, dst, ta1, td1, mv1, zrows_e)
    h2, ts2, td2, ta2, mx2 = _k2(acc1, ace1, h1, W_l1, b_l1, b_g1, W_g2,
                                 a_src2.reshape(-1), a_dst2.reshape(-1), S, R)
    mv2 = _mvec(mx2)
    acc2 = _sc_edge_pass(src, dst, ts2, td2, mv2, zrows)
    ace2 = _sc_e_pass(src, dst, ta2, td2, mv2, zrows_e)
    y = _k3(acc2, ace2, h1, h2, W_l2, b_l2, b_g2, R, W_fc, b_fc, W_out, b_out)
    return y[0].reshape(N)
